# baseline (speedup 1.0000x reference)
import functools

import jax
import jax.numpy as jnp
from jax import lax
from jax.experimental import pallas as pl
from jax.experimental.pallas import tpu as pltpu
from jax.experimental.pallas import tpu_sc as plsc

B = 4096
L = 200
D = 64
NSHAPE = 26
NCOLOUR = 10
NCOMBO = NSHAPE * NCOLOUR

NC = 2
NS = 16
NW = NC * NS
COLS_PER_W = B // NW
KGROUPS = COLS_PER_W // 16

TUNROLL = 16


def _hist_body(shapes_hbm, colours_hbm, hist_hbm, sblk, cblk, bins, sem_s, sem_c):
    wid = lax.axis_index("s") * NC + lax.axis_index("c")
    col0 = wid * COLS_PER_W

    cp_s = pltpu.async_copy(shapes_hbm.at[:, pl.ds(col0, COLS_PER_W)], sblk, sem_s)
    cp_c = pltpu.async_copy(colours_hbm.at[:, pl.ds(col0, COLS_PER_W)], cblk, sem_c)

    lane = jnp.arange(16, dtype=jnp.int32)
    zero16 = jnp.zeros((16,), jnp.float32)
    ones16 = jnp.ones((16,), jnp.float32)

    @plsc.parallel_loop(0, NCOMBO, 1, unroll=4)
    def _zero(c):
        for u in range(KGROUPS):
            bins[c, pl.ds(u * 16, 16)] = zero16

    cp_s.wait()
    cp_c.wait()

    def kbody(k, carry):
        blv = lane + k * 16
        koff = k * 16

        @plsc.parallel_loop(0, L, 1, unroll=TUNROLL)
        def _tok(t):
            s = sblk[t, pl.ds(koff, 16)]
            c = cblk[t, pl.ds(koff, 16)]
            plsc.addupdate_scatter(bins, [s * NCOLOUR + c, blv], ones16)

        return carry

    lax.fori_loop(0, KGROUPS, kbody, 0)

    pltpu.sync_copy(bins, hist_hbm.at[:, pl.ds(col0, COLS_PER_W)])


def _histogram(shapes_t, colours_t):
    mesh = plsc.VectorSubcoreMesh(core_axis_name="c", subcore_axis_name="s")
    hist = pl.kernel(
        _hist_body,
        mesh=mesh,
        compiler_params=pltpu.CompilerParams(
            needs_layout_passes=False,
        ),
        out_type=jax.ShapeDtypeStruct((NCOMBO, B), jnp.float32),
        scratch_types=[
            pltpu.VMEM((L, COLS_PER_W), jnp.int32),
            pltpu.VMEM((L, COLS_PER_W), jnp.int32),
            pltpu.VMEM((NCOMBO, COLS_PER_W), jnp.float32),
            pltpu.SemaphoreType.DMA,
            pltpu.SemaphoreType.DMA,
        ],
    )(shapes_t, colours_t)
    return hist


def _table_body(srep_ref, ctile_ref, wff_ref, bff_ref, t_ref):
    e = srep_ref[...] + ctile_ref[...]
    t_ref[...] = jax.nn.relu(
        jnp.dot(e, wff_ref[...], preferred_element_type=jnp.float32)
        + bff_ref[...]
    )


def _dense_body(hist_ref, t_ref, sym_ref, w1a_ref, w1b_ref, b1_ref,
                w2_ref, b2_ref, out_ref):
    seq = jnp.einsum('cb,cd->bd', hist_ref[...], t_ref[...],
                     preferred_element_type=jnp.float32) * (1.0 / L)
    h = jax.nn.relu(
        jnp.dot(seq, w1a_ref[...], preferred_element_type=jnp.float32)
        + jnp.dot(sym_ref[...], w1b_ref[...], preferred_element_type=jnp.float32)
        + b1_ref[...]
    )
    out_ref[...] = (
        jnp.dot(h, w2_ref[...], preferred_element_type=jnp.float32) + b2_ref[...]
    )


def kernel(shapes_list, colours_list, sym, shape_embed, colour_embed,
           W_ff, b_ff, W1, b1, W2, b2):
    hist = _histogram(shapes_list.T.astype(jnp.int32),
                      colours_list.T.astype(jnp.int32))

    s_rep = jnp.repeat(shape_embed, NCOLOUR, axis=0)
    c_tile = jnp.tile(colour_embed, (NSHAPE, 1))
    w1a = W1[:D, :]
    w1b = W1[D:, :]

    t_tab = pl.pallas_call(
        _table_body,
        out_shape=jax.ShapeDtypeStruct((NCOMBO, D), jnp.float32),
    )(s_rep, c_tile, W_ff, b_ff)

    nblk = 2
    bblk = B // nblk
    full = lambda shape: pl.BlockSpec(shape, lambda i: (0,) * len(shape))
    logits = pl.pallas_call(
        _dense_body,
        grid=(nblk,),
        in_specs=[
            pl.BlockSpec((NCOMBO, bblk), lambda i: (0, i)),
            full((NCOMBO, D)),
            pl.BlockSpec((bblk, 3), lambda i: (i, 0)),
            full((D, D)),
            full((3, D)),
            full((D,)),
            full((D, 2)),
            full((2,)),
        ],
        out_specs=pl.BlockSpec((bblk, 2), lambda i: (i, 0)),
        out_shape=jax.ShapeDtypeStruct((B, 2), jnp.float32),
        compiler_params=pltpu.CompilerParams(
            fuse_transposed_lhs_in_matmul=True,
        ),
    )(hist, t_tab, sym, w1a, w1b, b1, W2, b2)
    return logits


# speedup vs baseline: 1.1000x; 1.1000x over previous
import functools

import jax
import jax.numpy as jnp
from jax import lax
from jax.experimental import pallas as pl
from jax.experimental.pallas import tpu as pltpu
from jax.experimental.pallas import tpu_sc as plsc

B = 4096
L = 200
D = 64
NSHAPE = 26
NCOLOUR = 10
NCOMBO = NSHAPE * NCOLOUR

NC = 2
NS = 16
NW = NC * NS
COLS_PER_W = B // NW
KGROUPS = COLS_PER_W // 16

TUNROLL = 16


def _hist_body(shapes_hbm, colours_hbm, hist_hbm, sblk, cblk, bins, sem_s, sem_c):
    wid = lax.axis_index("s") * NC + lax.axis_index("c")
    col0 = wid * COLS_PER_W

    cp_s = pltpu.async_copy(shapes_hbm.at[:, pl.ds(col0, COLS_PER_W)], sblk, sem_s)
    cp_c = pltpu.async_copy(colours_hbm.at[:, pl.ds(col0, COLS_PER_W)], cblk, sem_c)

    lane = jnp.arange(16, dtype=jnp.int32)
    zero16 = jnp.zeros((16,), jnp.float32)
    ones16 = jnp.ones((16,), jnp.float32)

    @plsc.parallel_loop(0, NCOMBO, 1, unroll=4)
    def _zero(c):
        for u in range(KGROUPS):
            bins[c, pl.ds(u * 16, 16)] = zero16

    cp_s.wait()
    cp_c.wait()

    def kbody(k, carry):
        blv = lane + k * 16
        koff = k * 16

        @plsc.parallel_loop(0, L, 1, unroll=TUNROLL)
        def _tok(t):
            s = sblk[t, pl.ds(koff, 16)]
            c = cblk[t, pl.ds(koff, 16)]
            plsc.addupdate_scatter(bins, [s * NCOLOUR + c, blv], ones16)

        return carry

    lax.fori_loop(0, KGROUPS, kbody, 0)

    pltpu.sync_copy(bins, hist_hbm.at[:, pl.ds(col0, COLS_PER_W)])


def _histogram(shapes_t, colours_t):
    mesh = plsc.VectorSubcoreMesh(core_axis_name="c", subcore_axis_name="s")
    hist = pl.kernel(
        _hist_body,
        mesh=mesh,
        compiler_params=pltpu.CompilerParams(
            needs_layout_passes=False,
        ),
        out_type=jax.ShapeDtypeStruct((NCOMBO, B), jnp.float32),
        scratch_types=[
            pltpu.VMEM((L, COLS_PER_W), jnp.int32),
            pltpu.VMEM((L, COLS_PER_W), jnp.int32),
            pltpu.VMEM((NCOMBO, COLS_PER_W), jnp.float32),
            pltpu.SemaphoreType.DMA,
            pltpu.SemaphoreType.DMA,
        ],
    )(shapes_t, colours_t)
    return hist


def _table_body(srep_ref, ctile_ref, wff_ref, bff_ref, t_ref):
    e = srep_ref[...] + ctile_ref[...]
    t_ref[...] = jax.nn.relu(
        jnp.dot(e, wff_ref[...], preferred_element_type=jnp.float32)
        + bff_ref[...]
    )


def _dense_body(hist_ref, t_ref, sym_ref, w1a_ref, w1b_ref, b1_ref,
                w2t_ref, b2_ref, out_ref):
    seq = jnp.einsum('cb,cd->bd', hist_ref[...], t_ref[...],
                     preferred_element_type=jnp.float32) * (1.0 / L)
    h = jax.nn.relu(
        jnp.dot(seq, w1a_ref[...], preferred_element_type=jnp.float32)
        + jnp.dot(sym_ref[...], w1b_ref[...], preferred_element_type=jnp.float32)
        + b1_ref[...]
    )
    out_ref[...] = (
        jnp.einsum('ed,bd->eb', w2t_ref[...], h,
                   preferred_element_type=jnp.float32) + b2_ref[...][:, None]
    )


def kernel(shapes_list, colours_list, sym, shape_embed, colour_embed,
           W_ff, b_ff, W1, b1, W2, b2):
    hist = _histogram(shapes_list.T.astype(jnp.int32),
                      colours_list.T.astype(jnp.int32))

    s_rep = jnp.repeat(shape_embed, NCOLOUR, axis=0)
    c_tile = jnp.tile(colour_embed, (NSHAPE, 1))
    w1a = W1[:D, :]
    w1b = W1[D:, :]

    t_tab = pl.pallas_call(
        _table_body,
        out_shape=jax.ShapeDtypeStruct((NCOMBO, D), jnp.float32),
    )(s_rep, c_tile, W_ff, b_ff)

    nblk = 2
    bblk = B // nblk
    full = lambda shape: pl.BlockSpec(shape, lambda i: (0,) * len(shape))
    logits = pl.pallas_call(
        _dense_body,
        grid=(nblk,),
        in_specs=[
            pl.BlockSpec((NCOMBO, bblk), lambda i: (0, i)),
            full((NCOMBO, D)),
            pl.BlockSpec((bblk, 3), lambda i: (i, 0)),
            full((D, D)),
            full((3, D)),
            full((D,)),
            full((2, D)),
            full((2,)),
        ],
        out_specs=pl.BlockSpec((2, bblk), lambda i: (0, i)),
        out_shape=jax.ShapeDtypeStruct((2, B), jnp.float32),
        compiler_params=pltpu.CompilerParams(
            fuse_transposed_lhs_in_matmul=True,
        ),
    )(hist, t_tab, sym, w1a, w1b, b1, W2.T, b2)
    return logits.T
